# baseline (speedup 1.0000x reference)
import functools

import jax
import jax.numpy as jnp
from jax import lax
from jax.experimental import pallas as pl
from jax.experimental.pallas import tpu as pltpu
from jax.experimental.pallas import tpu_sc as plsc

EMB = 64
F0, F1, F2 = 5, 6, 2
NROWS = F0 * F1 * F2
N_EDGES = 800000
LANES = 16

_info = plsc.get_sparse_core_info()
NC = _info.num_cores
NS = _info.num_subcores
NW = NC * NS
PER_W = N_EDGES // NW
CHUNK = 128
N_FULL = PER_W // CHUNK
TAIL = PER_W - N_FULL * CHUNK

_mesh = plsc.VectorSubcoreMesh(core_axis_name="c", subcore_axis_name="s")


def _fused_table_body(w0_ref, w1_ref, w2_ref, out_ref):
    r = lax.broadcasted_iota(jnp.int32, (NROWS, F0), 0)
    c = lax.broadcasted_iota(jnp.int32, (NROWS, F0), 1)
    o0 = (r // (F1 * F2) == c).astype(jnp.float32)
    r = lax.broadcasted_iota(jnp.int32, (NROWS, F1), 0)
    c = lax.broadcasted_iota(jnp.int32, (NROWS, F1), 1)
    o1 = ((r // F2) % F1 == c).astype(jnp.float32)
    r = lax.broadcasted_iota(jnp.int32, (NROWS, F2), 0)
    c = lax.broadcasted_iota(jnp.int32, (NROWS, F2), 1)
    o2 = (r % F2 == c).astype(jnp.float32)
    hp = lax.Precision.HIGHEST
    t = (jnp.dot(o0, w0_ref[...], preferred_element_type=jnp.float32, precision=hp)
         + jnp.dot(o1, w1_ref[...], preferred_element_type=jnp.float32, precision=hp)
         + jnp.dot(o2, w2_ref[...], preferred_element_type=jnp.float32, precision=hp))
    out_ref[:, :EMB] = t
    out_ref[:, EMB:] = jnp.zeros((NROWS, EMB), jnp.float32)


_fused_table = pl.pallas_call(
    _fused_table_body,
    out_shape=jax.ShapeDtypeStruct((NROWS, 2 * EMB), jnp.float32),
)


@functools.partial(
    pl.kernel,
    mesh=_mesh,
    out_type=jax.ShapeDtypeStruct((N_EDGES, EMB), jnp.float32),
    scratch_types=[
        pltpu.VMEM((CHUNK,), jnp.int32),
        pltpu.VMEM((CHUNK,), jnp.int32),
        pltpu.VMEM((CHUNK,), jnp.int32),
        pltpu.VMEM((CHUNK,), jnp.int32),
        pltpu.VMEM((CHUNK, 2 * EMB), jnp.float32),
        pltpu.VMEM((CHUNK, EMB), jnp.float32),
        pltpu.SemaphoreType.DMA,
    ],
)
def _lookup(e0_hbm, e1_hbm, e2_hbm, tp_hbm, out_hbm,
            e0_v, e1_v, e2_v, idx_v, rows_v, rows64_v, sem):
    wid = lax.axis_index("s") * NC + lax.axis_index("c")
    wbase = wid * PER_W

    def compute_indices():
        for g in range(CHUNK // LANES):
            sl = pl.ds(g * LANES, LANES)
            c = e0_v[sl] * (F1 * F2) + e1_v[sl] * F2 + e2_v[sl]
            idx_v[sl] = jnp.minimum(jnp.maximum(c, 0), NROWS - 1)

    def strip_pad(nrows):
        for r in range(nrows):
            for h in range(EMB // LANES):
                sl = pl.ds(h * LANES, LANES)
                rows64_v[r, sl] = rows_v[r, sl]

    def load_cols(base, size):
        pltpu.sync_copy(e0_hbm.at[pl.ds(base, size)], e0_v.at[pl.ds(0, size)])
        pltpu.sync_copy(e1_hbm.at[pl.ds(base, size)], e1_v.at[pl.ds(0, size)])
        pltpu.sync_copy(e2_hbm.at[pl.ds(base, size)], e2_v.at[pl.ds(0, size)])

    def body(t, carry):
        base = wbase + t * CHUNK
        load_cols(base, CHUNK)
        compute_indices()
        pltpu.async_copy(tp_hbm.at[idx_v], rows_v, sem).wait()
        strip_pad(CHUNK)
        pltpu.sync_copy(rows64_v, out_hbm.at[pl.ds(base, CHUNK)])
        return carry

    lax.fori_loop(0, N_FULL, body, 0)

    tbase = wbase + N_FULL * CHUNK
    load_cols(tbase, TAIL)
    compute_indices()
    pltpu.async_copy(tp_hbm.at[idx_v], rows_v, sem).wait()
    strip_pad(TAIL)
    pltpu.sync_copy(rows64_v.at[pl.ds(0, TAIL)], out_hbm.at[pl.ds(tbase, TAIL)])


def kernel(edge_attr, W0, W1, W2):
    ea = edge_attr.astype(jnp.int32)
    tp = _fused_table(W0, W1, W2)
    return _lookup(ea[:, 0], ea[:, 1], ea[:, 2], tp)


# speedup vs baseline: 1.4316x; 1.4316x over previous
import functools

import jax
import jax.numpy as jnp
from jax import lax
from jax.experimental import pallas as pl
from jax.experimental.pallas import tpu as pltpu
from jax.experimental.pallas import tpu_sc as plsc

EMB = 64
F0, F1, F2 = 5, 6, 2
NROWS = F0 * F1 * F2
N_EDGES = 800000
LANES = 16
HREG = EMB // LANES

_info = plsc.get_sparse_core_info()
NC = _info.num_cores
NS = _info.num_subcores
NW = NC * NS
PER_W = N_EDGES // NW
CHUNK = 512
N_FULL = PER_W // CHUNK
TAIL = PER_W - N_FULL * CHUNK
GROUPS = CHUNK // LANES
TAIL_GROUPS = (TAIL + LANES - 1) // LANES

_mesh = plsc.VectorSubcoreMesh(core_axis_name="c", subcore_axis_name="s")


def _fused_table_body(w0_ref, w1_ref, w2_ref, out_ref):
    r = lax.broadcasted_iota(jnp.int32, (NROWS, F0), 0)
    c = lax.broadcasted_iota(jnp.int32, (NROWS, F0), 1)
    o0 = (r // (F1 * F2) == c).astype(jnp.float32)
    r = lax.broadcasted_iota(jnp.int32, (NROWS, F1), 0)
    c = lax.broadcasted_iota(jnp.int32, (NROWS, F1), 1)
    o1 = ((r // F2) % F1 == c).astype(jnp.float32)
    r = lax.broadcasted_iota(jnp.int32, (NROWS, F2), 0)
    c = lax.broadcasted_iota(jnp.int32, (NROWS, F2), 1)
    o2 = (r % F2 == c).astype(jnp.float32)
    hp = lax.Precision.HIGHEST
    out_ref[...] = (
        jnp.dot(o0, w0_ref[...], preferred_element_type=jnp.float32, precision=hp)
        + jnp.dot(o1, w1_ref[...], preferred_element_type=jnp.float32, precision=hp)
        + jnp.dot(o2, w2_ref[...], preferred_element_type=jnp.float32, precision=hp))


_fused_table = pl.pallas_call(
    _fused_table_body,
    out_shape=jax.ShapeDtypeStruct((NROWS, EMB), jnp.float32),
)


@functools.partial(
    pl.kernel,
    mesh=_mesh,
    out_type=jax.ShapeDtypeStruct((N_EDGES, EMB), jnp.float32),
    scratch_types=[
        pltpu.VMEM((NROWS, EMB), jnp.float32),
        pltpu.VMEM((3 * CHUNK,), jnp.int32),
        pltpu.VMEM((CHUNK, EMB), jnp.float32),
        pltpu.SemaphoreType.DMA,
    ],
)
def _lookup(ea_hbm, tp_hbm, out_hbm, t_v, ea_v, rows_v, sem):
    wid = lax.axis_index("s") * NC + lax.axis_index("c")
    wbase = wid * PER_W
    pltpu.sync_copy(tp_hbm, t_v)

    def assemble_group(g, carry):
        v = [ea_v[pl.ds(48 * g + 16 * j, LANES)] for j in range(3)]
        for k in range(LANES):
            a0 = v[(3 * k) // LANES][(3 * k) % LANES]
            a1 = v[(3 * k + 1) // LANES][(3 * k + 1) % LANES]
            a2 = v[(3 * k + 2) // LANES][(3 * k + 2) % LANES]
            c = a0 * (F1 * F2) + a1 * F2 + a2
            c = jnp.minimum(jnp.maximum(c, 0), NROWS - 1)
            row = g * LANES + k
            for h in range(HREG):
                sl = pl.ds(h * LANES, LANES)
                rows_v[row, sl] = t_v[c, sl]
        return carry

    def do_chunk(base, nrows, ngroups):
        pltpu.sync_copy(ea_hbm.at[pl.ds(base * 3, nrows * 3)],
                        ea_v.at[pl.ds(0, nrows * 3)])
        lax.fori_loop(0, ngroups, assemble_group, 0)
        pltpu.sync_copy(rows_v.at[pl.ds(0, nrows)],
                        out_hbm.at[pl.ds(base, nrows)])

    def body(t, carry):
        do_chunk(wbase + t * CHUNK, CHUNK, GROUPS)
        return carry

    lax.fori_loop(0, N_FULL, body, 0)
    do_chunk(wbase + N_FULL * CHUNK, TAIL, TAIL_GROUPS)


def kernel(edge_attr, W0, W1, W2):
    ea = edge_attr.astype(jnp.int32).reshape(-1)
    tp = _fused_table(W0, W1, W2)
    return _lookup(ea, tp)
